# baseline (speedup 1.0000x reference)
import jax
import jax.numpy as jnp
from jax import lax
from jax.experimental import pallas as pl
from jax.experimental.pallas import tpu as pltpu
from jax.experimental.pallas import tpu_sc as plsc

_N = 10000
_E = 320000
_D = 128
_NLAYERS = 4
_NC = 2
_NS = 16
_NW = _NC * _NS
_C = 128
_PER_TILE = _E // _NW
_GRP = 10
_ITERS = 8 * _GRP
_CAP = _ITERS * _C
_NPAD = 10240
_RPT = _NPAD // _NS

_mesh = plsc.VectorSubcoreMesh(core_axis_name="c", subcore_axis_name="s")


def _deg_kernel_body(col_hbm, out_hbm, col_v, ones_v, zbuf, deg_sh):
    cid = lax.axis_index("c")
    sid = lax.axis_index("s")
    wid = sid * _NC + cid
    pltpu.sync_copy(col_hbm.at[wid], col_v)
    one16 = jnp.ones((16,), jnp.float32)
    zero16 = jnp.zeros((16,), jnp.float32)
    for j in range(_C // 16):
        ones_v[pl.ds(j * 16, 16)] = one16
    for j in range(_RPT // 16):
        zbuf[pl.ds(j * 16, 16)] = zero16
    pltpu.sync_copy(zbuf, deg_sh.at[pl.ds(sid * _RPT, _RPT)])
    plsc.subcore_barrier()

    def body(it, carry):
        pltpu.sync_copy(ones_v, deg_sh.at[col_v.at[it]], add=True)
        return carry

    lax.fori_loop(0, _ITERS, body, 0)
    plsc.subcore_barrier()
    pltpu.sync_copy(deg_sh.at[pl.ds(sid * _RPT, _RPT)],
                    out_hbm.at[cid, pl.ds(sid * _RPT, _RPT)])


def _deg_call(col_slab):
    return pl.kernel(
        _deg_kernel_body,
        out_type=jax.ShapeDtypeStruct((_NC, _NPAD), jnp.float32),
        mesh=_mesh,
        scratch_types=[
            pltpu.VMEM((_ITERS, _C), jnp.int32),
            pltpu.VMEM((_C,), jnp.float32),
            pltpu.VMEM((_RPT,), jnp.float32),
            pltpu.VMEM_SHARED((_NPAD,), jnp.float32),
        ],
    )(col_slab)


_SLOT_R = (0, 16)
_SLOT_C = (8, 24)


def _msg_kernel_body(g_hbm, row_hbm, col_hbm, out_hbm,
                     idxb, buf0, buf1, s_sh, sem0, sem1, semi0, semi1,
                     sems0, sems1):
    cid = lax.axis_index("c")
    sid = lax.axis_index("s")
    wid = sid * _NC + cid

    bufs = (buf0, buf1)
    sems = (sem0, sem1)
    semis = (semi0, semi1)
    semss = (sems0, sems1)

    def idx_load(g_idx, slot, sem):
        pltpu.async_copy(row_hbm.at[wid, pl.ds(g_idx * 8, 8)],
                         idxb.at[pl.ds(_SLOT_R[slot], 8)], sem)
        pltpu.async_copy(col_hbm.at[wid, pl.ds(g_idx * 8, 8)],
                         idxb.at[pl.ds(_SLOT_C[slot], 8)], sem)

    idx_load(0, 0, semi0)
    idx_load(1, 1, semi1)

    zero16 = jnp.zeros((16,), jnp.float32)

    def zrow(i, carry):
        for j in range(_D // 16):
            buf0[i, pl.ds(j * 16, 16)] = zero16
        return carry

    lax.fori_loop(0, _C, zrow, 0)
    for k in range(_RPT // _C):
        pltpu.sync_copy(buf0, s_sh.at[pl.ds(sid * _RPT + k * _C, _C)])

    pltpu.make_async_copy(row_hbm.at[wid, pl.ds(0, 8)],
                          idxb.at[pl.ds(_SLOT_R[0], 8)], semi0).wait()
    pltpu.make_async_copy(col_hbm.at[wid, pl.ds(0, 8)],
                          idxb.at[pl.ds(_SLOT_C[0], 8)], semi0).wait()
    pltpu.async_copy(g_hbm.at[idxb.at[_SLOT_R[0]]], buf0, sem0)
    plsc.subcore_barrier()

    def group(g_idx, slot):
        nslot = 1 - slot
        for j in range(8):
            bufA, semA = bufs[j % 2], sems[j % 2]
            bufB, semB = bufs[1 - j % 2], sems[1 - j % 2]
            rrow = _SLOT_R[slot] + j
            crow = _SLOT_C[slot] + j
            pcrow = crow - 1 if j > 0 else _SLOT_C[nslot] + 7
            pltpu.make_async_copy(g_hbm.at[idxb.at[rrow]], bufA, semA).wait()
            pltpu.async_copy(bufA, s_sh.at[idxb.at[crow]], semss[j % 2],
                             add=True)

            def _wait_prev_scatter():
                pltpu.make_async_copy(bufB, s_sh.at[idxb.at[pcrow]],
                                      semss[1 - j % 2]).wait()

            if j < 7:
                if j == 0:
                    @pl.when(g_idx > 0)
                    def _():
                        _wait_prev_scatter()
                else:
                    _wait_prev_scatter()
                pltpu.async_copy(g_hbm.at[idxb.at[rrow + 1]], bufB, semB)
            else:
                _wait_prev_scatter()

                @pl.when(g_idx + 1 < _GRP)
                def _():
                    pltpu.make_async_copy(
                        row_hbm.at[wid, pl.ds((g_idx + 1) * 8, 8)],
                        idxb.at[pl.ds(_SLOT_R[nslot], 8)], semis[nslot]).wait()
                    pltpu.make_async_copy(
                        col_hbm.at[wid, pl.ds((g_idx + 1) * 8, 8)],
                        idxb.at[pl.ds(_SLOT_C[nslot], 8)], semis[nslot]).wait()
                    pltpu.async_copy(g_hbm.at[idxb.at[_SLOT_R[nslot]]],
                                     bufB, semB)
        @pl.when(g_idx + 2 < _GRP)
        def _():
            idx_load(g_idx + 2, slot, semis[slot])

    def pair(gi, carry):
        group(2 * gi, 0)
        group(2 * gi + 1, 1)
        return carry

    lax.fori_loop(0, _GRP // 2, pair, 0)
    pltpu.make_async_copy(bufs[1], s_sh.at[idxb.at[_SLOT_C[1] + 7]],
                          semss[1]).wait()
    plsc.subcore_barrier()
    pltpu.sync_copy(s_sh.at[pl.ds(sid * _RPT, _RPT)],
                    out_hbm.at[cid, pl.ds(sid * _RPT, _RPT)])


def _msg_call(g, row_slab, col_slab):
    f32 = jnp.float32
    return pl.kernel(
        _msg_kernel_body,
        out_type=jax.ShapeDtypeStruct((_NC, _NPAD, _D), f32),
        mesh=_mesh,
        scratch_types=[
            pltpu.VMEM((32, _C), jnp.int32),
            pltpu.VMEM((_C, _D), f32),
            pltpu.VMEM((_C, _D), f32),
            pltpu.VMEM_SHARED((_NPAD, _D), f32),
            pltpu.SemaphoreType.DMA,
            pltpu.SemaphoreType.DMA,
            pltpu.SemaphoreType.DMA,
            pltpu.SemaphoreType.DMA,
            pltpu.SemaphoreType.DMA,
            pltpu.SemaphoreType.DMA,
        ],
    )(g, row_slab, col_slab)


_BN = 1000


def _pre_body(x, d0, d1, wpT, bp, wiT, bi, h0_out, g_out, dis_out):
    deg = d0[0] + d1[0] + 1.0
    dis = lax.rsqrt(deg)
    h = jax.nn.relu(jnp.dot(x[...], wpT[...],
                            preferred_element_type=jnp.float32) + bp[...])
    h0 = jax.nn.relu(jnp.dot(h, wiT[...],
                             preferred_element_type=jnp.float32) + bi[...])
    h0_out[...] = h0
    g_out[...] = dis * h0
    dis_out[...] = dis


def _pre_call(x, d0, d1, wpT, bp, wiT, bi):
    f32 = jnp.float32
    row_spec = pl.BlockSpec((_BN, _D), lambda i: (i, 0))
    one_spec = pl.BlockSpec((_BN, 1), lambda i: (i, 0))
    w_spec = pl.BlockSpec((_D, _D), lambda i: (0, 0))
    b_spec = pl.BlockSpec((1, _D), lambda i: (0, 0))
    d0_spec = pl.BlockSpec((1, _BN, 1), lambda i: (0, i, 0))
    d1_spec = pl.BlockSpec((1, _BN, 1), lambda i: (1, i, 0))
    return pl.pallas_call(
        _pre_body,
        grid=(_N // _BN,),
        in_specs=[row_spec, d0_spec, d1_spec, w_spec, b_spec, w_spec, b_spec],
        out_specs=[row_spec, row_spec, one_spec],
        out_shape=[jax.ShapeDtypeStruct((_N, _D), f32),
                   jax.ShapeDtypeStruct((_N, _D), f32),
                   jax.ShapeDtypeStruct((_N, 1), f32)],
    )(x, d0, d1, wpT, bp, wiT, bi)


def _gru_core(hv, disv, s0, s1, wzh, wza, wrh, wra, whh, wha, bz, br, bh):
    aggr = disv * (disv * hv + s0 + s1)
    dot = lambda a, b: jnp.dot(a, b, preferred_element_type=jnp.float32)
    z = jax.nn.sigmoid(dot(hv, wzh) + dot(aggr, wza) + bz)
    r = jax.nn.sigmoid(dot(hv, wrh) + dot(aggr, wra) + br)
    hc = jax.nn.relu(dot(r * hv, whh) + dot(aggr, wha) + bh)
    return (1.0 - z) * hv + z * hc


def _gru_body(h, s0, s1, dis, wzh, wza, wrh, wra, whh, wha,
              bz, br, bh, h_out, g_out):
    disv = dis[...]
    hn = _gru_core(h[...], disv, s0[0], s1[0], wzh[...], wza[...], wrh[...],
                   wra[...], whh[...], wha[...], bz[...], br[...], bh[...])
    h_out[...] = hn
    g_out[...] = disv * hn


def _gru_call(h, s2, dis, wzh, wza, wrh, wra, whh, wha, bz, br, bh):
    f32 = jnp.float32
    row_spec = pl.BlockSpec((_BN, _D), lambda i: (i, 0))
    one_spec = pl.BlockSpec((_BN, 1), lambda i: (i, 0))
    w_spec = pl.BlockSpec((_D, _D), lambda i: (0, 0))
    b_spec = pl.BlockSpec((1, _D), lambda i: (0, 0))
    s0_spec = pl.BlockSpec((1, _BN, _D), lambda i: (0, i, 0))
    s1_spec = pl.BlockSpec((1, _BN, _D), lambda i: (1, i, 0))
    return pl.pallas_call(
        _gru_body,
        grid=(_N // _BN,),
        in_specs=[row_spec, s0_spec, s1_spec, one_spec,
                  w_spec, w_spec, w_spec, w_spec, w_spec, w_spec,
                  b_spec, b_spec, b_spec],
        out_specs=[row_spec, row_spec],
        out_shape=[jax.ShapeDtypeStruct((_N, _D), f32),
                   jax.ShapeDtypeStruct((_N, _D), f32)],
    )(h, s2, s2, dis, wzh, wza, wrh, wra, whh, wha, bz, br, bh)


def _gru_fin_body(h, h0, s0, s1, dis, wzh, wza, wrh, wra, whh, wha,
                  bz, br, bh, w1a, w1b, b1, w2, b2, out):
    hn = _gru_core(h[...], dis[...], s0[0], s1[0], wzh[...], wza[...],
                   wrh[...], wra[...], whh[...], wha[...], bz[...], br[...],
                   bh[...])
    dot = lambda a, b: jnp.dot(a, b, preferred_element_type=jnp.float32)
    xo = jax.nn.relu(dot(h0[...], w1a[...]) + dot(hn, w1b[...]) + b1[...])
    out[...] = dot(xo, w2[...]) + b2[...]


def _gru_fin_call(h, h0, s2, dis, wzh, wza, wrh, wra, whh, wha, bz, br, bh,
                  w1a, w1b, b1, w2, b2):
    row_spec = pl.BlockSpec((_BN, _D), lambda i: (i, 0))
    one_spec = pl.BlockSpec((_BN, 1), lambda i: (i, 0))
    w_spec = pl.BlockSpec((_D, _D), lambda i: (0, 0))
    b_spec = pl.BlockSpec((1, _D), lambda i: (0, 0))
    s0_spec = pl.BlockSpec((1, _BN, _D), lambda i: (0, i, 0))
    s1_spec = pl.BlockSpec((1, _BN, _D), lambda i: (1, i, 0))
    return pl.pallas_call(
        _gru_fin_body,
        grid=(_N // _BN,),
        in_specs=[row_spec, row_spec, s0_spec, s1_spec, one_spec,
                  w_spec, w_spec, w_spec, w_spec, w_spec, w_spec,
                  b_spec, b_spec, b_spec,
                  w_spec, w_spec, b_spec, w_spec, b_spec],
        out_specs=row_spec,
        out_shape=jax.ShapeDtypeStruct((_N, _D), jnp.float32),
    )(h, h0, s2, s2, dis, wzh, wza, wrh, wra, whh, wha, bz, br, bh,
      w1a, w1b, b1, w2, b2)


def kernel(x, edge_index, preproc_W, preproc_b, init_W, init_b,
           Wz, bz, Wr, br, Wh, bh, fc1_W, fc1_b, fc2_W, fc2_b):
    row = edge_index[0].reshape(_NW, _PER_TILE)
    col = edge_index[1].reshape(_NW, _PER_TILE)
    padn = _CAP - _PER_TILE
    k = jnp.arange(padn, dtype=jnp.int32)[None, :]
    w = jnp.arange(_NW, dtype=jnp.int32)[:, None]
    pad_read = (w * 37 + k * 13) % _N
    pad_write = _N + (w * 7 + k) % (_NPAD - _N)
    row_slab = jnp.concatenate([row, pad_read], axis=1).reshape(_NW, _ITERS, _C)
    col_slab = jnp.concatenate([col, pad_write], axis=1).reshape(_NW, _ITERS, _C)

    deg2 = _deg_call(col_slab).reshape(_NC, _NPAD, 1)

    h0, g, dis = _pre_call(x, deg2, deg2, preproc_W.T,
                           preproc_b.reshape(1, _D), init_W.T,
                           init_b.reshape(1, _D))

    wzh, wza = Wz[:, :_D].T, Wz[:, _D:].T
    wrh, wra = Wr[:, :_D].T, Wr[:, _D:].T
    whh, wha = Wh[:, :_D].T, Wh[:, _D:].T
    bz2, br2, bh2 = bz.reshape(1, _D), br.reshape(1, _D), bh.reshape(1, _D)

    h = h0
    for _ in range(_NLAYERS - 1):
        s2 = _msg_call(g, row_slab, col_slab)
        h, g = _gru_call(h, s2, dis,
                         wzh, wza, wrh, wra, whh, wha, bz2, br2, bh2)

    w2 = jnp.pad(fc2_W.T, ((0, 0), (0, _D - 2)))
    b2 = jnp.pad(fc2_b, (0, _D - 2)).reshape(1, _D)
    s2 = _msg_call(g, row_slab, col_slab)
    out_pad = _gru_fin_call(h, h0, s2, dis,
                            wzh, wza, wrh, wra, whh, wha, bz2, br2, bh2,
                            fc1_W[:, :_D].T, fc1_W[:, _D:].T,
                            fc1_b.reshape(1, _D), w2, b2)
    return out_pad[:, :2]


# speedup vs baseline: 1.0271x; 1.0271x over previous
import jax
import jax.numpy as jnp
from jax import lax
from jax.experimental import pallas as pl
from jax.experimental.pallas import tpu as pltpu
from jax.experimental.pallas import tpu_sc as plsc

_N = 10000
_E = 320000
_D = 128
_NLAYERS = 4
_NC = 2
_NS = 16
_NW = _NC * _NS
_C = 128
_PER_TILE = _E // _NW
_GRP = 10
_ITERS = 8 * _GRP
_CAP = _ITERS * _C
_NPAD = 10240
_RPT = _NPAD // _NS

_mesh = plsc.VectorSubcoreMesh(core_axis_name="c", subcore_axis_name="s")


def _deg_kernel_body(col_hbm, out_hbm, col_v, ones_v, zbuf, deg_sh):
    cid = lax.axis_index("c")
    sid = lax.axis_index("s")
    wid = sid * _NC + cid
    pltpu.sync_copy(col_hbm.at[wid], col_v)
    one16 = jnp.ones((16,), jnp.float32)
    zero16 = jnp.zeros((16,), jnp.float32)
    for j in range(_C // 16):
        ones_v[pl.ds(j * 16, 16)] = one16
    for j in range(_RPT // 16):
        zbuf[pl.ds(j * 16, 16)] = zero16
    pltpu.sync_copy(zbuf, deg_sh.at[pl.ds(sid * _RPT, _RPT)])
    plsc.subcore_barrier()

    def body(it, carry):
        pltpu.sync_copy(ones_v, deg_sh.at[col_v.at[it]], add=True)
        return carry

    lax.fori_loop(0, _ITERS, body, 0)
    plsc.subcore_barrier()
    pltpu.sync_copy(deg_sh.at[pl.ds(sid * _RPT, _RPT)],
                    out_hbm.at[cid, pl.ds(sid * _RPT, _RPT)])


def _deg_call(col_slab):
    return pl.kernel(
        _deg_kernel_body,
        out_type=jax.ShapeDtypeStruct((_NC, _NPAD), jnp.float32),
        mesh=_mesh,
        scratch_types=[
            pltpu.VMEM((_ITERS, _C), jnp.int32),
            pltpu.VMEM((_C,), jnp.float32),
            pltpu.VMEM((_RPT,), jnp.float32),
            pltpu.VMEM_SHARED((_NPAD,), jnp.float32),
        ],
    )(col_slab)


_SLOT_R = (0, 16)
_SLOT_C = (8, 24)


def _msg_kernel_body(g_hbm, row_hbm, col_hbm, out_hbm,
                     idxb, buf0, buf1, s_sh, sem0, sem1, semi0, semi1,
                     sems0, sems1):
    cid = lax.axis_index("c")
    sid = lax.axis_index("s")
    wid = sid * _NC + cid

    bufs = (buf0, buf1)
    sems = (sem0, sem1)
    semis = (semi0, semi1)
    semss = (sems0, sems1)

    def idx_load(g_idx, slot, sem):
        pltpu.async_copy(row_hbm.at[wid, pl.ds(g_idx * 8, 8)],
                         idxb.at[pl.ds(_SLOT_R[slot], 8)], sem)
        pltpu.async_copy(col_hbm.at[wid, pl.ds(g_idx * 8, 8)],
                         idxb.at[pl.ds(_SLOT_C[slot], 8)], sem)

    idx_load(0, 0, semi0)
    idx_load(1, 1, semi1)

    zero16 = jnp.zeros((16,), jnp.float32)

    def zrow(i, carry):
        for j in range(_D // 16):
            buf0[i, pl.ds(j * 16, 16)] = zero16
        return carry

    lax.fori_loop(0, _C, zrow, 0)
    for k in range(_RPT // _C):
        pltpu.sync_copy(buf0, s_sh.at[pl.ds(sid * _RPT + k * _C, _C)])

    pltpu.make_async_copy(row_hbm.at[wid, pl.ds(0, 8)],
                          idxb.at[pl.ds(_SLOT_R[0], 8)], semi0).wait()
    pltpu.make_async_copy(col_hbm.at[wid, pl.ds(0, 8)],
                          idxb.at[pl.ds(_SLOT_C[0], 8)], semi0).wait()
    pltpu.async_copy(g_hbm.at[idxb.at[_SLOT_R[0]]], buf0, sem0)
    plsc.subcore_barrier()

    def group(g_idx, slot):
        nslot = 1 - slot
        for j in range(8):
            bufA, semA = bufs[j % 2], sems[j % 2]
            bufB, semB = bufs[1 - j % 2], sems[1 - j % 2]
            rrow = _SLOT_R[slot] + j
            crow = _SLOT_C[slot] + j
            pcrow = crow - 1 if j > 0 else _SLOT_C[nslot] + 7
            pltpu.make_async_copy(g_hbm.at[idxb.at[rrow]], bufA, semA).wait()
            pltpu.async_copy(bufA, s_sh.at[idxb.at[crow]], semss[j % 2],
                             add=True)

            def _wait_prev_scatter():
                pltpu.make_async_copy(bufB, s_sh.at[idxb.at[pcrow]],
                                      semss[1 - j % 2]).wait()

            if j < 7:
                if j == 0:
                    @pl.when(g_idx > 0)
                    def _():
                        _wait_prev_scatter()
                else:
                    _wait_prev_scatter()
                pltpu.async_copy(g_hbm.at[idxb.at[rrow + 1]], bufB, semB)
            else:
                _wait_prev_scatter()

                @pl.when(g_idx + 1 < _GRP)
                def _():
                    pltpu.make_async_copy(
                        row_hbm.at[wid, pl.ds((g_idx + 1) * 8, 8)],
                        idxb.at[pl.ds(_SLOT_R[nslot], 8)], semis[nslot]).wait()
                    pltpu.make_async_copy(
                        col_hbm.at[wid, pl.ds((g_idx + 1) * 8, 8)],
                        idxb.at[pl.ds(_SLOT_C[nslot], 8)], semis[nslot]).wait()
                    pltpu.async_copy(g_hbm.at[idxb.at[_SLOT_R[nslot]]],
                                     bufB, semB)
        @pl.when(g_idx + 2 < _GRP)
        def _():
            idx_load(g_idx + 2, slot, semis[slot])

    def pair(gi, carry):
        group(2 * gi, 0)
        group(2 * gi + 1, 1)
        return carry

    lax.fori_loop(0, _GRP // 2, pair, 0)
    pltpu.make_async_copy(bufs[1], s_sh.at[idxb.at[_SLOT_C[1] + 7]],
                          semss[1]).wait()
    plsc.subcore_barrier()
    pltpu.sync_copy(s_sh.at[pl.ds(sid * _RPT, _RPT)],
                    out_hbm.at[cid, pl.ds(sid * _RPT, _RPT)])


def _msg_call(g, row_slab, col_slab):
    f32 = jnp.float32
    return pl.kernel(
        _msg_kernel_body,
        out_type=jax.ShapeDtypeStruct((_NC, _NPAD, _D), f32),
        mesh=_mesh,
        scratch_types=[
            pltpu.VMEM((32, _C), jnp.int32),
            pltpu.VMEM((_C, _D), f32),
            pltpu.VMEM((_C, _D), f32),
            pltpu.VMEM_SHARED((_NPAD, _D), f32),
            pltpu.SemaphoreType.DMA,
            pltpu.SemaphoreType.DMA,
            pltpu.SemaphoreType.DMA,
            pltpu.SemaphoreType.DMA,
            pltpu.SemaphoreType.DMA,
            pltpu.SemaphoreType.DMA,
        ],
    )(g, row_slab, col_slab)


_BN = 2000


def _pre_body(x, d0, d1, wpT, bp, wiT, bi, h0_out, g_out, dis_out):
    deg = d0[0] + d1[0] + 1.0
    dis = lax.rsqrt(deg)
    h = jax.nn.relu(jnp.dot(x[...], wpT[...],
                            preferred_element_type=jnp.float32) + bp[...])
    h0 = jax.nn.relu(jnp.dot(h, wiT[...],
                             preferred_element_type=jnp.float32) + bi[...])
    h0_out[...] = h0
    g_out[...] = dis * h0
    dis_out[...] = dis


def _pre_call(x, d0, d1, wpT, bp, wiT, bi):
    f32 = jnp.float32
    row_spec = pl.BlockSpec((_BN, _D), lambda i: (i, 0))
    one_spec = pl.BlockSpec((_BN, 1), lambda i: (i, 0))
    w_spec = pl.BlockSpec((_D, _D), lambda i: (0, 0))
    b_spec = pl.BlockSpec((1, _D), lambda i: (0, 0))
    d0_spec = pl.BlockSpec((1, _BN, 1), lambda i: (0, i, 0))
    d1_spec = pl.BlockSpec((1, _BN, 1), lambda i: (1, i, 0))
    return pl.pallas_call(
        _pre_body,
        grid=(_N // _BN,),
        in_specs=[row_spec, d0_spec, d1_spec, w_spec, b_spec, w_spec, b_spec],
        out_specs=[row_spec, row_spec, one_spec],
        out_shape=[jax.ShapeDtypeStruct((_N, _D), f32),
                   jax.ShapeDtypeStruct((_N, _D), f32),
                   jax.ShapeDtypeStruct((_N, 1), f32)],
    )(x, d0, d1, wpT, bp, wiT, bi)


def _gru_core(hv, disv, s0, s1, wzh, wza, wrh, wra, whh, wha, bz, br, bh):
    aggr = disv * (disv * hv + s0 + s1)
    dot = lambda a, b: jnp.dot(a, b, preferred_element_type=jnp.float32)
    z = jax.nn.sigmoid(dot(hv, wzh) + dot(aggr, wza) + bz)
    r = jax.nn.sigmoid(dot(hv, wrh) + dot(aggr, wra) + br)
    hc = jax.nn.relu(dot(r * hv, whh) + dot(aggr, wha) + bh)
    return (1.0 - z) * hv + z * hc


def _gru_body(h, s0, s1, dis, wzh, wza, wrh, wra, whh, wha,
              bz, br, bh, h_out, g_out):
    disv = dis[...]
    hn = _gru_core(h[...], disv, s0[0], s1[0], wzh[...], wza[...], wrh[...],
                   wra[...], whh[...], wha[...], bz[...], br[...], bh[...])
    h_out[...] = hn
    g_out[...] = disv * hn


def _gru_call(h, s2, dis, wzh, wza, wrh, wra, whh, wha, bz, br, bh):
    f32 = jnp.float32
    row_spec = pl.BlockSpec((_BN, _D), lambda i: (i, 0))
    one_spec = pl.BlockSpec((_BN, 1), lambda i: (i, 0))
    w_spec = pl.BlockSpec((_D, _D), lambda i: (0, 0))
    b_spec = pl.BlockSpec((1, _D), lambda i: (0, 0))
    s0_spec = pl.BlockSpec((1, _BN, _D), lambda i: (0, i, 0))
    s1_spec = pl.BlockSpec((1, _BN, _D), lambda i: (1, i, 0))
    return pl.pallas_call(
        _gru_body,
        grid=(_N // _BN,),
        in_specs=[row_spec, s0_spec, s1_spec, one_spec,
                  w_spec, w_spec, w_spec, w_spec, w_spec, w_spec,
                  b_spec, b_spec, b_spec],
        out_specs=[row_spec, row_spec],
        out_shape=[jax.ShapeDtypeStruct((_N, _D), f32),
                   jax.ShapeDtypeStruct((_N, _D), f32)],
    )(h, s2, s2, dis, wzh, wza, wrh, wra, whh, wha, bz, br, bh)


def _gru_fin_body(h, h0, s0, s1, dis, wzh, wza, wrh, wra, whh, wha,
                  bz, br, bh, w1a, w1b, b1, w2, b2, out):
    hn = _gru_core(h[...], dis[...], s0[0], s1[0], wzh[...], wza[...],
                   wrh[...], wra[...], whh[...], wha[...], bz[...], br[...],
                   bh[...])
    dot = lambda a, b: jnp.dot(a, b, preferred_element_type=jnp.float32)
    xo = jax.nn.relu(dot(h0[...], w1a[...]) + dot(hn, w1b[...]) + b1[...])
    out[...] = dot(xo, w2[...]) + b2[...]


def _gru_fin_call(h, h0, s2, dis, wzh, wza, wrh, wra, whh, wha, bz, br, bh,
                  w1a, w1b, b1, w2, b2):
    row_spec = pl.BlockSpec((_BN, _D), lambda i: (i, 0))
    one_spec = pl.BlockSpec((_BN, 1), lambda i: (i, 0))
    w_spec = pl.BlockSpec((_D, _D), lambda i: (0, 0))
    b_spec = pl.BlockSpec((1, _D), lambda i: (0, 0))
    s0_spec = pl.BlockSpec((1, _BN, _D), lambda i: (0, i, 0))
    s1_spec = pl.BlockSpec((1, _BN, _D), lambda i: (1, i, 0))
    return pl.pallas_call(
        _gru_fin_body,
        grid=(_N // _BN,),
        in_specs=[row_spec, row_spec, s0_spec, s1_spec, one_spec,
                  w_spec, w_spec, w_spec, w_spec, w_spec, w_spec,
                  b_spec, b_spec, b_spec,
                  w_spec, w_spec, b_spec, w_spec, b_spec],
        out_specs=row_spec,
        out_shape=jax.ShapeDtypeStruct((_N, _D), jnp.float32),
    )(h, h0, s2, s2, dis, wzh, wza, wrh, wra, whh, wha, bz, br, bh,
      w1a, w1b, b1, w2, b2)


def kernel(x, edge_index, preproc_W, preproc_b, init_W, init_b,
           Wz, bz, Wr, br, Wh, bh, fc1_W, fc1_b, fc2_W, fc2_b):
    row = edge_index[0].reshape(_NW, _PER_TILE)
    col = edge_index[1].reshape(_NW, _PER_TILE)
    padn = _CAP - _PER_TILE
    k = jnp.arange(padn, dtype=jnp.int32)[None, :]
    w = jnp.arange(_NW, dtype=jnp.int32)[:, None]
    pad_read = (w * 37 + k * 13) % _N
    pad_write = _N + (w * 7 + k) % (_NPAD - _N)
    row_slab = jnp.concatenate([row, pad_read], axis=1).reshape(_NW, _ITERS, _C)
    col_slab = jnp.concatenate([col, pad_write], axis=1).reshape(_NW, _ITERS, _C)

    deg2 = _deg_call(col_slab).reshape(_NC, _NPAD, 1)

    h0, g, dis = _pre_call(x, deg2, deg2, preproc_W.T,
                           preproc_b.reshape(1, _D), init_W.T,
                           init_b.reshape(1, _D))

    wzh, wza = Wz[:, :_D].T, Wz[:, _D:].T
    wrh, wra = Wr[:, :_D].T, Wr[:, _D:].T
    whh, wha = Wh[:, :_D].T, Wh[:, _D:].T
    bz2, br2, bh2 = bz.reshape(1, _D), br.reshape(1, _D), bh.reshape(1, _D)

    h = h0
    for _ in range(_NLAYERS - 1):
        s2 = _msg_call(g, row_slab, col_slab)
        h, g = _gru_call(h, s2, dis,
                         wzh, wza, wrh, wra, whh, wha, bz2, br2, bh2)

    w2 = jnp.pad(fc2_W.T, ((0, 0), (0, _D - 2)))
    b2 = jnp.pad(fc2_b, (0, _D - 2)).reshape(1, _D)
    s2 = _msg_call(g, row_slab, col_slab)
    out_pad = _gru_fin_call(h, h0, s2, dis,
                            wzh, wza, wrh, wra, whh, wha, bz2, br2, bh2,
                            fc1_W[:, :_D].T, fc1_W[:, _D:].T,
                            fc1_b.reshape(1, _D), w2, b2)
    return out_pad[:, :2]
